# Initial kernel scaffold; baseline (speedup 1.0000x reference)
import jax
import jax.numpy as jnp
from jax.experimental import pallas as pl


def kernel(x, edge_index, edge_attr, u, batch, W1, b1, W2, b2):
    raise NotImplementedError("write your pallas kernel here")


# speedup vs baseline: 2.7603x; 2.7603x over previous
import functools

import jax
import jax.numpy as jnp
from jax import lax
from jax.experimental import pallas as pl
from jax.experimental.pallas import tpu as pltpu
from jax.experimental.pallas import tpu_sc as plsc

N_NODES = 100000
N_EDGES = 1600000
HID = 32

NC = 2
NS = 16
HALF = N_NODES // NC
TRASH = HALF
ACC_ROWS = HALF + 48
ZROWS = ACC_ROWS // NS
WFULL = 3080
WTAIL = ZROWS - WFULL

EB = 512
NBLK = N_EDGES // EB
BPT = -(-NBLK // NS)


def _sc_segment_sum(col, edge_attr, zeros):
    mesh = plsc.VectorSubcoreMesh(core_axis_name="c", subcore_axis_name="s")

    @functools.partial(
        pl.kernel,
        out_type=jax.ShapeDtypeStruct((N_NODES, HID), jnp.float32),
        mesh=mesh,
        scratch_types=[
            pltpu.VMEM_SHARED((ACC_ROWS, HID), jnp.float32),
            pltpu.VMEM((EB,), jnp.int32),
            pltpu.VMEM((EB, HID), jnp.float32),
            pltpu.VMEM((EB // 128, 128), jnp.int32),
        ],
        compiler_params=pltpu.CompilerParams(use_tc_tiling_on_sc=False),
    )
    def k(col_hbm, edge_hbm, zeros_hbm, out_hbm, acc, colv, rows, idxb):
        c = lax.axis_index("c")
        s = lax.axis_index("s")
        base = c * HALF

        pltpu.sync_copy(zeros_hbm.at[pl.ds(s * ZROWS, ZROWS)],
                        acc.at[pl.ds(s * ZROWS, ZROWS)])
        plsc.subcore_barrier()

        lo = jnp.minimum(s * BPT, NBLK)
        hi = jnp.minimum((s + 1) * BPT, NBLK)

        def blk(b, carry):
            off = b * EB
            pltpu.sync_copy(col_hbm.at[pl.ds(off, EB)], colv)
            pltpu.sync_copy(edge_hbm.at[pl.ds(off, EB)], rows)
            for j in range(EB // 16):
                v = colv[pl.ds(j * 16, 16)]
                rel = v - base
                m = (rel >= 0) & (rel < HALF)
                idxb[j // 8, pl.ds((j % 8) * 16, 16)] = jnp.where(m, rel, TRASH)
            for t in range(EB // 128):
                pltpu.sync_copy(rows.at[pl.ds(t * 128, 128)],
                                acc.at[idxb.at[t]], add=True)
            return carry

        lax.fori_loop(lo, hi, blk, 0)
        plsc.subcore_barrier()

        pltpu.sync_copy(acc.at[pl.ds(s * ZROWS, WFULL)],
                        out_hbm.at[pl.ds(base + s * ZROWS, WFULL)])

        @pl.when(s < NS - 1)
        def _():
            pltpu.sync_copy(acc.at[pl.ds(s * ZROWS + WFULL, WTAIL)],
                            out_hbm.at[pl.ds(base + s * ZROWS + WFULL, WTAIL)])

    return k(col, edge_attr, zeros)


RBLK = 2000


def _mlp_body(x_ref, agg_ref, w1x_ref, w1a_ref, b1_ref, w2_ref, b2_ref, o_ref):
    h = jnp.dot(x_ref[...], w1x_ref[...], preferred_element_type=jnp.float32)
    h = h + jnp.dot(agg_ref[...], w1a_ref[...], preferred_element_type=jnp.float32)
    h = jnp.maximum(h + b1_ref[...], 0.0)
    h = jnp.dot(h, w2_ref[...], preferred_element_type=jnp.float32) + b2_ref[...]
    o_ref[...] = jnp.maximum(h, 0.0)


def _tc_mlp(x, agg, w1x, w1a, b1, w2, b2):
    nin = x.shape[1]
    grid = (N_NODES // RBLK,)
    return pl.pallas_call(
        _mlp_body,
        grid=grid,
        in_specs=[
            pl.BlockSpec((RBLK, nin), lambda i: (i, 0)),
            pl.BlockSpec((RBLK, HID), lambda i: (i, 0)),
            pl.BlockSpec((nin, HID), lambda i: (0, 0)),
            pl.BlockSpec((HID, HID), lambda i: (0, 0)),
            pl.BlockSpec((1, HID), lambda i: (0, 0)),
            pl.BlockSpec((HID, HID), lambda i: (0, 0)),
            pl.BlockSpec((1, HID), lambda i: (0, 0)),
        ],
        out_specs=pl.BlockSpec((RBLK, HID), lambda i: (i, 0)),
        out_shape=jax.ShapeDtypeStruct((N_NODES, HID), jnp.float32),
    )(x, agg, w1x, w1a, b1, w2, b2)


def kernel(x, edge_index, edge_attr, u, batch, W1, b1, W2, b2):
    col = edge_index[1].astype(jnp.int32)
    zeros = jnp.zeros((ACC_ROWS, HID), jnp.float32)
    agg = _sc_segment_sum(col, edge_attr, zeros)
    w1x = W1[: x.shape[1]]
    w1a = W1[x.shape[1]:]
    return _tc_mlp(x, agg, w1x, w1a, b1.reshape(1, HID), W2, b2.reshape(1, HID))


# speedup vs baseline: 4.7031x; 1.7038x over previous
import functools

import jax
import jax.numpy as jnp
from jax import lax
from jax.experimental import pallas as pl
from jax.experimental.pallas import tpu as pltpu
from jax.experimental.pallas import tpu_sc as plsc

N_NODES = 100000
N_EDGES = 1600000
HID = 32

NC = 2
NS = 16
FH = HID // NC
EB = 640
CROWS = EB // 128
NBLK = N_EDGES // EB
BPT = -(-NBLK // NS)
ZR = N_NODES // NS


def _sc_segment_sum(col2d, edge_attr, zeros):
    mesh = plsc.VectorSubcoreMesh(core_axis_name="c", subcore_axis_name="s")

    @functools.partial(
        pl.kernel,
        out_type=jax.ShapeDtypeStruct((NC, N_NODES, FH), jnp.float32),
        mesh=mesh,
        scratch_types=[
            pltpu.VMEM_SHARED((N_NODES, FH), jnp.float32),
            pltpu.VMEM((CROWS, 128), jnp.int32),
            pltpu.VMEM((CROWS, 128), jnp.int32),
            pltpu.VMEM((EB, FH), jnp.float32),
            pltpu.VMEM((EB, FH), jnp.float32),
            pltpu.SemaphoreType.DMA,
            pltpu.SemaphoreType.DMA,
            pltpu.SemaphoreType.DMA,
        ],
        compiler_params=pltpu.CompilerParams(use_tc_tiling_on_sc=False),
    )
    def k(col_hbm, edge_hbm, zeros_hbm, out_hbm,
          acc, colv0, colv1, rows0, rows1, fs0, fs1, ssem):
        c = lax.axis_index("c")
        s = lax.axis_index("s")
        fbase = c * FH

        pltpu.sync_copy(zeros_hbm.at[pl.ds(s * ZR, ZR)],
                        acc.at[pl.ds(s * ZR, ZR)])
        plsc.subcore_barrier()

        lo = jnp.minimum(s * BPT, NBLK)
        hi = jnp.minimum((s + 1) * BPT, NBLK)

        def fill(b, colv, rows, sem):
            pltpu.async_copy(col_hbm.at[pl.ds(b * CROWS, CROWS)], colv, sem)
            pltpu.async_copy(
                edge_hbm.at[pl.ds(b * EB, EB), pl.ds(fbase, FH)], rows, sem)

        def wait_fill(colv, rows, sem):
            pltpu.make_async_copy(
                col_hbm.at[pl.ds(0, CROWS)], colv, sem).wait()
            pltpu.make_async_copy(
                edge_hbm.at[pl.ds(0, EB), pl.ds(0, FH)], rows, sem).wait()

        def process(colv, rows):
            for t in range(CROWS):
                pltpu.sync_copy(rows.at[pl.ds(t * 128, 128)],
                                acc.at[colv.at[t]], add=True)

        @pl.when(lo < hi)
        def _():
            fill(lo, colv0, rows0, fs0)

        def blk(b, carry):
            even = (b - lo) % 2 == 0

            @pl.when((b + 1 < hi) & even)
            def _():
                fill(b + 1, colv1, rows1, fs1)

            @pl.when((b + 1 < hi) & jnp.logical_not(even))
            def _():
                fill(b + 1, colv0, rows0, fs0)

            @pl.when(even)
            def _():
                wait_fill(colv0, rows0, fs0)
                process(colv0, rows0)

            @pl.when(jnp.logical_not(even))
            def _():
                wait_fill(colv1, rows1, fs1)
                process(colv1, rows1)

            return carry

        lax.fori_loop(lo, hi, blk, 0)
        plsc.subcore_barrier()

        pltpu.sync_copy(acc.at[pl.ds(s * ZR, ZR)],
                        out_hbm.at[c, pl.ds(s * ZR, ZR)])

    return k(col2d, edge_attr, zeros)


RBLK = 2000


def _mlp_body(x_ref, agg_ref, w1x_ref, w1a_ref, b1_ref, w2_ref, b2_ref, o_ref):
    a = agg_ref[...]
    h = jnp.dot(x_ref[...], w1x_ref[...], preferred_element_type=jnp.float32)
    h = h + jnp.dot(a[0], w1a_ref[0], preferred_element_type=jnp.float32)
    h = h + jnp.dot(a[1], w1a_ref[1], preferred_element_type=jnp.float32)
    h = jnp.maximum(h + b1_ref[...], 0.0)
    h = jnp.dot(h, w2_ref[...], preferred_element_type=jnp.float32) + b2_ref[...]
    o_ref[...] = jnp.maximum(h, 0.0)


def _tc_mlp(x, agg, w1x, w1a, b1, w2, b2):
    nin = x.shape[1]
    grid = (N_NODES // RBLK,)
    return pl.pallas_call(
        _mlp_body,
        grid=grid,
        in_specs=[
            pl.BlockSpec((RBLK, nin), lambda i: (i, 0)),
            pl.BlockSpec((NC, RBLK, FH), lambda i: (0, i, 0)),
            pl.BlockSpec((nin, HID), lambda i: (0, 0)),
            pl.BlockSpec((NC, FH, HID), lambda i: (0, 0, 0)),
            pl.BlockSpec((1, HID), lambda i: (0, 0)),
            pl.BlockSpec((HID, HID), lambda i: (0, 0)),
            pl.BlockSpec((1, HID), lambda i: (0, 0)),
        ],
        out_specs=pl.BlockSpec((RBLK, HID), lambda i: (i, 0)),
        out_shape=jax.ShapeDtypeStruct((N_NODES, HID), jnp.float32),
    )(x, agg, w1x, w1a, b1, w2, b2)


def kernel(x, edge_index, edge_attr, u, batch, W1, b1, W2, b2):
    nin = x.shape[1]
    col2d = edge_index[1].astype(jnp.int32).reshape(N_EDGES // 128, 128)
    zeros = jnp.zeros((N_NODES, FH), jnp.float32)
    agg = _sc_segment_sum(col2d, edge_attr, zeros)
    w1x = W1[:nin]
    w1a = W1[nin:].reshape(NC, FH, HID)
    return _tc_mlp(x, agg, w1x, w1a, b1.reshape(1, HID), W2, b2.reshape(1, HID))


# speedup vs baseline: 4.7959x; 1.0197x over previous
import functools

import jax
import jax.numpy as jnp
from jax import lax
from jax.experimental import pallas as pl
from jax.experimental.pallas import tpu as pltpu
from jax.experimental.pallas import tpu_sc as plsc

N_NODES = 100000
N_EDGES = 1600000
HID = 32

NC = 2
NS = 16
FH = HID // NC
EB = 640
CROWS = EB // 128
NBLK = N_EDGES // EB
BPT = -(-NBLK // NS)
ZR = N_NODES // NS
ZCH = 250


def _sc_segment_sum(col, edge_attr):
    mesh = plsc.VectorSubcoreMesh(core_axis_name="c", subcore_axis_name="s")

    @functools.partial(
        pl.kernel,
        out_type=jax.ShapeDtypeStruct((NC, N_NODES, FH), jnp.float32),
        mesh=mesh,
        scratch_types=[
            pltpu.VMEM_SHARED((N_NODES, FH), jnp.float32),
            pltpu.VMEM((CROWS, 128), jnp.int32),
            pltpu.VMEM((CROWS, 128), jnp.int32),
            pltpu.VMEM((EB, FH), jnp.float32),
            pltpu.VMEM((EB, FH), jnp.float32),
            pltpu.VMEM((ZCH, FH), jnp.float32),
            pltpu.SemaphoreType.DMA,
            pltpu.SemaphoreType.DMA,
            pltpu.SemaphoreType.DMA,
        ],
        compiler_params=pltpu.CompilerParams(use_tc_tiling_on_sc=False),
    )
    def k(col_hbm, edge_hbm, out_hbm,
          acc, colv0, colv1, rows0, rows1, zbuf, fs0, fs1, ssem):
        c = lax.axis_index("c")
        s = lax.axis_index("s")
        fbase = c * FH

        zv = jnp.zeros((16,), jnp.float32)

        def zrow(i, carry):
            zbuf[i, :] = zv
            return carry

        lax.fori_loop(0, ZCH, zrow, 0)
        for z in range(ZR // ZCH):
            pltpu.sync_copy(zbuf, acc.at[pl.ds(s * ZR + z * ZCH, ZCH)])
        plsc.subcore_barrier()

        lo = jnp.minimum(s * BPT, NBLK)
        hi = jnp.minimum((s + 1) * BPT, NBLK)

        def fill(b, colv, rows, sem):
            for t in range(CROWS):
                pltpu.async_copy(
                    col_hbm.at[pl.ds(b * EB + t * 128, 128)], colv.at[t], sem)
            pltpu.async_copy(
                edge_hbm.at[pl.ds(b * EB, EB), pl.ds(fbase, FH)], rows, sem)

        def wait_fill(colv, rows, sem):
            for t in range(CROWS):
                pltpu.make_async_copy(
                    col_hbm.at[pl.ds(0, 128)], colv.at[t], sem).wait()
            pltpu.make_async_copy(
                edge_hbm.at[pl.ds(0, EB), pl.ds(0, FH)], rows, sem).wait()

        def process(colv, rows):
            for t in range(CROWS):
                pltpu.async_copy(rows.at[pl.ds(t * 128, 128)],
                                 acc.at[colv.at[t]], ssem, add=True)
            for t in range(CROWS):
                pltpu.make_async_copy(rows.at[pl.ds(t * 128, 128)],
                                      acc.at[colv.at[t]], ssem).wait()

        @pl.when(lo < hi)
        def _():
            fill(lo, colv0, rows0, fs0)

        def blk(b, carry):
            even = (b - lo) % 2 == 0

            @pl.when((b + 1 < hi) & even)
            def _():
                fill(b + 1, colv1, rows1, fs1)

            @pl.when((b + 1 < hi) & jnp.logical_not(even))
            def _():
                fill(b + 1, colv0, rows0, fs0)

            @pl.when(even)
            def _():
                wait_fill(colv0, rows0, fs0)
                process(colv0, rows0)

            @pl.when(jnp.logical_not(even))
            def _():
                wait_fill(colv1, rows1, fs1)
                process(colv1, rows1)

            return carry

        lax.fori_loop(lo, hi, blk, 0)
        plsc.subcore_barrier()

        pltpu.sync_copy(acc.at[pl.ds(s * ZR, ZR)],
                        out_hbm.at[c, pl.ds(s * ZR, ZR)])

    return k(col, edge_attr)


RBLK = 2000


def _mlp_body(x_ref, agg_ref, w1x_ref, w1a_ref, b1_ref, w2_ref, b2_ref, o_ref):
    a = agg_ref[...]
    h = jnp.dot(x_ref[...], w1x_ref[...], preferred_element_type=jnp.float32)
    h = h + jnp.dot(a[0], w1a_ref[0], preferred_element_type=jnp.float32)
    h = h + jnp.dot(a[1], w1a_ref[1], preferred_element_type=jnp.float32)
    h = jnp.maximum(h + b1_ref[...], 0.0)
    h = jnp.dot(h, w2_ref[...], preferred_element_type=jnp.float32) + b2_ref[...]
    o_ref[...] = jnp.maximum(h, 0.0)


def _tc_mlp(x, agg, w1x, w1a, b1, w2, b2):
    nin = x.shape[1]
    grid = (N_NODES // RBLK,)
    return pl.pallas_call(
        _mlp_body,
        grid=grid,
        in_specs=[
            pl.BlockSpec((RBLK, nin), lambda i: (i, 0)),
            pl.BlockSpec((NC, RBLK, FH), lambda i: (0, i, 0)),
            pl.BlockSpec((nin, HID), lambda i: (0, 0)),
            pl.BlockSpec((NC, FH, HID), lambda i: (0, 0, 0)),
            pl.BlockSpec((1, HID), lambda i: (0, 0)),
            pl.BlockSpec((HID, HID), lambda i: (0, 0)),
            pl.BlockSpec((1, HID), lambda i: (0, 0)),
        ],
        out_specs=pl.BlockSpec((RBLK, HID), lambda i: (i, 0)),
        out_shape=jax.ShapeDtypeStruct((N_NODES, HID), jnp.float32),
    )(x, agg, w1x, w1a, b1, w2, b2)


def kernel(x, edge_index, edge_attr, u, batch, W1, b1, W2, b2):
    nin = x.shape[1]
    col = edge_index[1].astype(jnp.int32)
    agg = _sc_segment_sum(col, edge_attr)
    w1x = W1[:nin]
    w1a = W1[nin:].reshape(NC, FH, HID)
    return _tc_mlp(x, agg, w1x, w1a, b1.reshape(1, HID), W2, b2.reshape(1, HID))
